# baseline (speedup 1.0000x reference)
import jax
import jax.numpy as jnp
from jax import lax
from jax.experimental import pallas as pl
from jax.experimental.pallas import tpu as pltpu
from jax.experimental.pallas import tpu_sc as plsc

NU = 5000
NS = 5000
D = 128
H = 128
E = 320000

NCORE = 2
NSUB = 16
CHUNK = 64
SUPER = 16
EP = 327680
PADN = EP - E
ROWS_CH = EP // CHUNK
W_CH = ROWS_CH // (NCORE * NSUB)
NSUPER = W_CH // SUPER
NAGG = 5120
HB = 3 * NAGG
HCHUNK = 128
HGROUP = 4
HEP = 3 * EP


def _mesh():
    return plsc.VectorSubcoreMesh(
        core_axis_name="c", subcore_axis_name="s",
        num_cores=NCORE, num_subcores=NSUB)


def _hist_body(zh, ones_h, hidx, out, hist, ibuf, ones, bounce, ssem, isem):
    c = lax.axis_index("c")
    s = lax.axis_index("s")
    w = c * NSUB + s
    z0 = s * (HB // NSUB)
    pltpu.sync_copy(zh.at[pl.ds(z0, HB // NSUB)], bounce)
    pltpu.sync_copy(bounce, hist.at[pl.ds(z0, HB // NSUB)])
    pltpu.sync_copy(ones_h, ones)
    plsc.subcore_barrier()
    w_rows = HEP // HCHUNK // (NCORE * NSUB)
    wrow = w * w_rows
    nsuper = w_rows // HGROUP

    def idx_dma(sp, q):
        return pltpu.make_async_copy(
            hidx.at[pl.ds(wrow + sp * HGROUP, HGROUP)], ibuf.at[q], isem)

    def sc_dma(q, p):
        return pltpu.make_async_copy(ones, hist.at[ibuf.at[q, p]], ssem)

    d0 = idx_dma(0, 0)
    d0.start()
    d0.wait()

    def super_loop(sp, carry):
        q = lax.rem(sp, 2)
        qn = lax.rem(sp + 1, 2)

        @pl.when(sp > 0)
        def _():
            idx_dma(sp, q).wait()
            for p in range(HGROUP):
                sc_dma(q, p).wait()

        @pl.when(sp < nsuper - 1)
        def _():
            idx_dma(sp + 1, qn).start()
        for p in range(HGROUP):
            pltpu.async_copy(ones, hist.at[ibuf.at[q, p]], ssem, add=True)
        return carry

    lax.fori_loop(0, nsuper, super_loop, 0)
    for p in range(HGROUP):
        sc_dma(lax.rem(nsuper - 1, 2), p).wait()
    plsc.subcore_barrier()
    pltpu.sync_copy(hist.at[pl.ds(z0, HB // NSUB)], bounce)
    pltpu.sync_copy(bounce, out.at[pl.ds(c * HB + z0, HB // NSUB)])


def _hist_call(zh, ones_h, hidx):
    return pl.kernel(
        _hist_body,
        out_type=jax.ShapeDtypeStruct((2 * HB,), jnp.float32),
        mesh=_mesh(),
        scratch_types=[
            pltpu.VMEM_SHARED((HB,), jnp.float32),
            pltpu.VMEM((2, HGROUP, HCHUNK), jnp.int32),
            pltpu.VMEM((HCHUNK,), jnp.float32),
            pltpu.VMEM((HB // NSUB,), jnp.float32),
            pltpu.SemaphoreType.DMA,
            pltpu.SemaphoreType.DMA,
        ],
    )(zh, ones_h, hidx)


def _seg_body(z, ya, yb, ga, sa, gb, sb, outa, outb,
              agga, aggb, iga, isa, igb, isb, rowsa, rowsb,
              isem, ga0, ga1, gb0, gb1, sa0, sa1, sb0, sb1):
    c = lax.axis_index("c")
    s = lax.axis_index("s")
    w = c * NSUB + s
    z0 = s * (NAGG // NSUB)
    pltpu.sync_copy(z.at[pl.ds(z0, NAGG // NSUB)], agga.at[pl.ds(z0, NAGG // NSUB)])
    pltpu.sync_copy(z.at[pl.ds(z0, NAGG // NSUB)], aggb.at[pl.ds(z0, NAGG // NSUB)])
    plsc.subcore_barrier()
    wrow = w * W_CH

    gsem = ((ga0, ga1), (gb0, gb1))
    ssem = ((sa0, sa1), (sb0, sb1))
    rows = (rowsa, rowsb)
    gi = (iga, igb)
    si = (isa, isb)
    ghbm = (ga, gb)
    shbm = (sa, sb)
    yy = (ya, yb)
    agg = (agga, aggb)

    def idx_copies(sp, q):
        rb = wrow + sp * SUPER
        return [pltpu.make_async_copy(ghbm[d].at[pl.ds(rb, SUPER)], gi[d].at[q], isem)
                for d in range(2)] + \
               [pltpu.make_async_copy(shbm[d].at[pl.ds(rb, SUPER)], si[d].at[q], isem)
                for d in range(2)]

    def fire_gather(d, q, k, p):
        return pltpu.async_copy(yy[d].at[gi[d].at[q, k]], rows[d].at[p],
                                gsem[d][p])

    def wait_gather(d, q, k, p):
        pltpu.make_async_copy(yy[d].at[gi[d].at[q, k]], rows[d].at[p],
                              gsem[d][p]).wait()

    def fire_scatter(d, q, k, p):
        return pltpu.async_copy(rows[d].at[p], agg[d].at[si[d].at[q, k]],
                                ssem[d][p], add=True)

    def wait_scatter(d, q, p):
        pltpu.make_async_copy(rows[d].at[p], agg[d].at[si[d].at[q, 0]],
                              ssem[d][p]).wait()

    for dsc in idx_copies(0, 0):
        dsc.start()
        dsc.wait()

    def outer(sp, carry):
        q = lax.rem(sp, 2)
        qn = lax.rem(sp + 1, 2)

        @pl.when(sp > 0)
        def _():
            for dsc in idx_copies(sp, q):
                dsc.wait()

        @pl.when(sp < NSUPER - 1)
        def _():
            for dsc in idx_copies(sp + 1, qn):
                dsc.start()

        def inner(kk, icarry):
            first = jnp.logical_and(sp == 0, kk == 0)
            for p in range(2):
                @pl.when(jnp.logical_not(first))
                def _():
                    for d in range(2):
                        wait_scatter(d, q, p)
                k = 2 * kk + p
                for d in range(2):
                    fire_gather(d, q, k, p)
            for p in range(2):
                k = 2 * kk + p
                for d in range(2):
                    wait_gather(d, q, k, p)
                    fire_scatter(d, q, k, p)
            return icarry

        lax.fori_loop(0, SUPER // 2, inner, 0)
        return carry

    lax.fori_loop(0, NSUPER, outer, 0)
    for d in range(2):
        wait_scatter(d, (NSUPER - 1) % 2, 0)
        wait_scatter(d, (NSUPER - 1) % 2, 1)
    plsc.subcore_barrier()
    b0 = s * 200
    pltpu.sync_copy(agga.at[pl.ds(b0, 200)], outa.at[pl.ds(c * NU + b0, 200)])
    pltpu.sync_copy(aggb.at[pl.ds(b0, 200)], outb.at[pl.ds(c * NU + b0, 200)])

    @pl.when(s < 9)
    def _():
        b1 = (s + NSUB) * 200
        pltpu.sync_copy(agga.at[pl.ds(b1, 200)], outa.at[pl.ds(c * NU + b1, 200)])
        pltpu.sync_copy(aggb.at[pl.ds(b1, 200)], outb.at[pl.ds(c * NU + b1, 200)])


def _seg_call(z, ya, yb, ga, sa, gb, sb):
    return pl.kernel(
        _seg_body,
        out_type=(jax.ShapeDtypeStruct((2 * NU, H), jnp.float32),
                  jax.ShapeDtypeStruct((2 * NU, H), jnp.float32)),
        mesh=_mesh(),
        scratch_types=[
            pltpu.VMEM_SHARED((NAGG, H), jnp.float32),
            pltpu.VMEM_SHARED((NAGG, H), jnp.float32),
            pltpu.VMEM((2, SUPER, CHUNK), jnp.int32),
            pltpu.VMEM((2, SUPER, CHUNK), jnp.int32),
            pltpu.VMEM((2, SUPER, CHUNK), jnp.int32),
            pltpu.VMEM((2, SUPER, CHUNK), jnp.int32),
            pltpu.VMEM((2, CHUNK, H), jnp.float32),
            pltpu.VMEM((2, CHUNK, H), jnp.float32),
        ] + [pltpu.SemaphoreType.DMA] * 9,
    )(z, ya, yb, ga, sa, gb, sb)


def _scales_body(h_ref, o_ref):
    h = h_ref[...]
    one = jnp.float32(1.0)
    du = h[0:40] + h[120:160]
    ds = h[40:80] + h[160:200]
    cu = h[80:120] + h[200:240]
    o_ref[0:40] = lax.rsqrt(jnp.maximum(du, one))
    o_ref[40:80] = lax.rsqrt(jnp.maximum(ds, one))
    o_ref[80:120] = one / jnp.maximum(cu, one)
    o_ref[120:160] = one / jnp.maximum(ds, one)


def _scales_call(hist):
    return pl.pallas_call(
        _scales_body,
        out_shape=jax.ShapeDtypeStruct((160, 128), jnp.float32),
    )(hist)


_BLK = 1000
_GRID = NU // _BLK


def _rowspec():
    return pl.BlockSpec((_BLK, H), lambda i: (i, 0))


def _colspec():
    return pl.BlockSpec((_BLK, 1), lambda i: (i, 0))


def _pspec():
    return pl.BlockSpec((2, _BLK, H), lambda i: (0, i, 0))


def _wspec():
    return pl.BlockSpec((H, H), lambda i: (0, 0))


def _lgcn_pre_body(u_ref, s_ref, ru_ref, rs_ref, ut_ref, st_ref):
    ut_ref[...] = u_ref[...] * ru_ref[...]
    st_ref[...] = s_ref[...] * rs_ref[...]


def _lgcn_pre_call(u, s, ru, rs):
    return pl.pallas_call(
        _lgcn_pre_body,
        grid=(_GRID,),
        in_specs=[_rowspec(), _rowspec(), _colspec(), _colspec()],
        out_specs=(_rowspec(), _rowspec()),
        out_shape=(jax.ShapeDtypeStruct((NU, H), jnp.float32),
                   jax.ShapeDtypeStruct((NS, H), jnp.float32)),
    )(u, s, ru, rs)


def _lgcn_post1_body(pu_ref, ps_ref, u0_ref, s0_ref, ru_ref, rs_ref,
                     au_ref, as_ref, ut_ref, st_ref):
    ru = ru_ref[...]
    rs = rs_ref[...]
    uo = (pu_ref[0] + pu_ref[1]) * ru
    so = (ps_ref[0] + ps_ref[1]) * rs
    au_ref[...] = u0_ref[...] + uo
    as_ref[...] = s0_ref[...] + so
    ut_ref[...] = uo * ru
    st_ref[...] = so * rs


def _lgcn_post1_call(pu, ps, u0, s0, ru, rs):
    return pl.pallas_call(
        _lgcn_post1_body,
        grid=(_GRID,),
        in_specs=[_pspec(), _pspec(), _rowspec(), _rowspec(),
                  _colspec(), _colspec()],
        out_specs=(_rowspec(), _rowspec(), _rowspec(), _rowspec()),
        out_shape=tuple(jax.ShapeDtypeStruct((NU, H), jnp.float32)
                        for _ in range(4)),
    )(pu, ps, u0, s0, ru, rs)


def _lgcn_post2_body(pu_ref, ps_ref, au_ref, as_ref, ru_ref, rs_ref,
                     cu_ref, cs_ref):
    third = jnp.float32(1.0 / 3.0)
    cu_ref[...] = (au_ref[...] + (pu_ref[0] + pu_ref[1]) * ru_ref[...]) * third
    cs_ref[...] = (as_ref[...] + (ps_ref[0] + ps_ref[1]) * rs_ref[...]) * third


def _lgcn_post2_call(pu, ps, au, as_, ru, rs):
    return pl.pallas_call(
        _lgcn_post2_body,
        grid=(_GRID,),
        in_specs=[_pspec(), _pspec(), _rowspec(), _rowspec(),
                  _colspec(), _colspec()],
        out_specs=(_rowspec(), _rowspec()),
        out_shape=(jax.ShapeDtypeStruct((NU, H), jnp.float32),
                   jax.ShapeDtypeStruct((NS, H), jnp.float32)),
    )(pu, ps, au, as_, ru, rs)


def _mm_body(xu_ref, xs_ref, wus_ref, wsu_ref, wfu_ref, wfs_ref,
             yu_ref, ys_ref, su_ref, ss_ref):
    xu = xu_ref[...]
    xs = xs_ref[...]
    dn = (((1,), (0,)), ((), ()))
    yu_ref[...] = lax.dot_general(xu, wus_ref[...], dn,
                                  preferred_element_type=jnp.float32)
    ys_ref[...] = lax.dot_general(xs, wsu_ref[...], dn,
                                  preferred_element_type=jnp.float32)
    su_ref[...] = lax.dot_general(xu, wfu_ref[...], dn,
                                  preferred_element_type=jnp.float32)
    ss_ref[...] = lax.dot_general(xs, wfs_ref[...], dn,
                                  preferred_element_type=jnp.float32)


def _mm_call(xu, xs, wus, wsu, wfu, wfs):
    return pl.pallas_call(
        _mm_body,
        grid=(_GRID,),
        in_specs=[_rowspec(), _rowspec(), _wspec(), _wspec(), _wspec(), _wspec()],
        out_specs=tuple(_rowspec() for _ in range(4)),
        out_shape=tuple(jax.ShapeDtypeStruct((NU, H), jnp.float32)
                        for _ in range(4)),
    )(xu, xs, wus, wsu, wfu, wfs)


def _postmm_body(su0_ref, ss0_ref, pu_ref, ps_ref, icu_ref, ics_ref,
                 wus_ref, wsu_ref, wfu_ref, wfs_ref,
                 yu_ref, ys_ref, su_ref, ss_ref):
    zero = jnp.float32(0.0)
    xu = jnp.maximum(su0_ref[...] + (pu_ref[0] + pu_ref[1]) * icu_ref[...], zero)
    xs = jnp.maximum(ss0_ref[...] + (ps_ref[0] + ps_ref[1]) * ics_ref[...], zero)
    dn = (((1,), (0,)), ((), ()))
    yu_ref[...] = lax.dot_general(xu, wus_ref[...], dn,
                                  preferred_element_type=jnp.float32)
    ys_ref[...] = lax.dot_general(xs, wsu_ref[...], dn,
                                  preferred_element_type=jnp.float32)
    su_ref[...] = lax.dot_general(xu, wfu_ref[...], dn,
                                  preferred_element_type=jnp.float32)
    ss_ref[...] = lax.dot_general(xs, wfs_ref[...], dn,
                                  preferred_element_type=jnp.float32)


def _postmm_call(su0, ss0, pu, ps, icu, ics, wus, wsu, wfu, wfs):
    return pl.pallas_call(
        _postmm_body,
        grid=(_GRID,),
        in_specs=[_rowspec(), _rowspec(), _pspec(), _pspec(),
                  _colspec(), _colspec(),
                  _wspec(), _wspec(), _wspec(), _wspec()],
        out_specs=tuple(_rowspec() for _ in range(4)),
        out_shape=tuple(jax.ShapeDtypeStruct((NU, H), jnp.float32)
                        for _ in range(4)),
    )(su0, ss0, pu, ps, icu, ics, wus, wsu, wfu, wfs)


def _final_body(su_ref, ss_ref, pu_ref, ps_ref, icu_ref, ics_ref,
                ucat_ref, scat_ref, uo_ref, so_ref):
    zero = jnp.float32(0.0)
    uo_ref[:, 0:H] = ucat_ref[...]
    so_ref[:, 0:H] = scat_ref[...]
    uo_ref[:, H:2 * H] = jnp.maximum(
        su_ref[...] + (pu_ref[0] + pu_ref[1]) * icu_ref[...], zero)
    so_ref[:, H:2 * H] = jnp.maximum(
        ss_ref[...] + (ps_ref[0] + ps_ref[1]) * ics_ref[...], zero)


def _final_call(su, ss, pu, ps, icu, ics, ucat, scat):
    wide = pl.BlockSpec((_BLK, 2 * H), lambda i: (i, 0))
    return pl.pallas_call(
        _final_body,
        grid=(_GRID,),
        in_specs=[_rowspec(), _rowspec(), _pspec(), _pspec(),
                  _colspec(), _colspec(), _rowspec(), _rowspec()],
        out_specs=(wide, wide),
        out_shape=(jax.ShapeDtypeStruct((NU, 2 * H), jnp.float32),
                   jax.ShapeDtypeStruct((NS, 2 * H), jnp.float32)),
    )(su, ss, pu, ps, icu, ics, ucat, scat)


def _pad_gather(idx):
    pad = (jnp.arange(PADN, dtype=jnp.int32) * 13) % NU
    return jnp.concatenate([idx.astype(jnp.int32), pad]).reshape(ROWS_CH, CHUNK)


def _pad_scatter(idx):
    pad = NU + (jnp.arange(PADN, dtype=jnp.int32) % (NAGG - NU))
    return jnp.concatenate([idx.astype(jnp.int32), pad]).reshape(ROWS_CH, CHUNK)


def _hist_idx(idx, h):
    pad = h * NAGG + NU + (jnp.arange(PADN, dtype=jnp.int32) % (NAGG - NU))
    return jnp.concatenate([idx.astype(jnp.int32) + h * NAGG, pad])


def kernel(user_x, spot_x, lgcn_user, lgcn_spot, edge_index_us, edge_index_su,
           W0_self_u, W0_self_s, W0_su, W0_us,
           W1_self_u, W1_self_s, W1_su, W1_us,
           W2_self_u, W2_self_s, W2_su, W2_us):
    z = jnp.zeros((NAGG, H), jnp.float32)
    zh = jnp.zeros((HB,), jnp.float32)
    ones_h = jnp.ones((HCHUNK,), jnp.float32)

    us0 = edge_index_us[0]
    us1 = edge_index_us[1]
    su0 = edge_index_su[0]
    su1 = edge_index_su[1]

    us_g0 = _pad_gather(us0)
    us_g1 = _pad_gather(us1)
    us_s0 = _pad_scatter(us0)
    us_s1 = _pad_scatter(us1)
    su_g0 = _pad_gather(su0)
    su_s1 = _pad_scatter(su1)

    hidx = jnp.concatenate([
        _hist_idx(us0, 0), _hist_idx(us1, 1), _hist_idx(su1, 2),
    ]).reshape(HEP // HCHUNK, HCHUNK)

    hist = _hist_call(zh, ones_h, hidx)
    scales = _scales_call(hist.reshape(240, 128))
    sc = scales.reshape(4, NAGG)[:, :NU]
    ru = sc[0][:, None]
    rs = sc[1][:, None]
    icu = sc[2][:, None]
    ics = sc[3][:, None]

    ut0, st0 = _lgcn_pre_call(lgcn_user, lgcn_spot, ru, rs)
    pa, pb = _seg_call(z, st0, ut0, us_g1, us_s0, us_g0, us_s1)
    acc_u, acc_s, ut1, st1 = _lgcn_post1_call(
        pa.reshape(2, NU, H), pb.reshape(2, NS, H), lgcn_user, lgcn_spot, ru, rs)
    pa, pb = _seg_call(z, st1, ut1, us_g1, us_s0, us_g0, us_s1)
    u_cat, s_cat = _lgcn_post2_call(
        pa.reshape(2, NU, H), pb.reshape(2, NS, H), acc_u, acc_s, ru, rs)

    yu, ys, su_t, ss_t = _mm_call(user_x, spot_x, W0_us, W0_su,
                                  W0_self_u, W0_self_s)
    ps_, pu_ = _seg_call(z, yu, ys, us_g0, us_s1, su_g0, su_s1)
    yu, ys, su_t, ss_t = _postmm_call(
        su_t, ss_t, pu_.reshape(2, NU, H), ps_.reshape(2, NS, H), icu, ics,
        W1_us, W1_su, W1_self_u, W1_self_s)
    ps_, pu_ = _seg_call(z, yu, ys, us_g0, us_s1, su_g0, su_s1)
    yu, ys, su_t, ss_t = _postmm_call(
        su_t, ss_t, pu_.reshape(2, NU, H), ps_.reshape(2, NS, H), icu, ics,
        W2_us, W2_su, W2_self_u, W2_self_s)
    ps_, pu_ = _seg_call(z, yu, ys, us_g0, us_s1, su_g0, su_s1)
    return _final_call(su_t, ss_t, pu_.reshape(2, NU, H),
                       ps_.reshape(2, NS, H), icu, ics, u_cat, s_cat)


# speedup vs baseline: 1.0116x; 1.0116x over previous
import jax
import jax.numpy as jnp
from jax import lax
from jax.experimental import pallas as pl
from jax.experimental.pallas import tpu as pltpu
from jax.experimental.pallas import tpu_sc as plsc

NU = 5000
NS = 5000
D = 128
H = 128
E = 320000

NCORE = 2
NSUB = 16
CHUNK = 64
SUPER = 16
EP = 327680
PADN = EP - E
ROWS_CH = EP // CHUNK
W_CH = ROWS_CH // (NCORE * NSUB)
NSUPER = W_CH // SUPER
NAGG = 5120
HB = 3 * NAGG
HCHUNK = 128
HGROUP = 4
HEP = 3 * EP


def _mesh():
    return plsc.VectorSubcoreMesh(
        core_axis_name="c", subcore_axis_name="s",
        num_cores=NCORE, num_subcores=NSUB)


def _hist_body(zh, ones_h, hidx, out, hist, ibuf, ones, bounce, ssem, isem):
    c = lax.axis_index("c")
    s = lax.axis_index("s")
    w = c * NSUB + s
    z0 = s * (HB // NSUB)
    pltpu.sync_copy(zh.at[pl.ds(z0, HB // NSUB)], bounce)
    pltpu.sync_copy(bounce, hist.at[pl.ds(z0, HB // NSUB)])
    pltpu.sync_copy(ones_h, ones)
    plsc.subcore_barrier()
    w_rows = HEP // HCHUNK // (NCORE * NSUB)
    wrow = w * w_rows
    nsuper = w_rows // HGROUP

    def idx_dma(sp, q):
        return pltpu.make_async_copy(
            hidx.at[pl.ds(wrow + sp * HGROUP, HGROUP)], ibuf.at[q], isem)

    def sc_dma(q, p):
        return pltpu.make_async_copy(ones, hist.at[ibuf.at[q, p]], ssem)

    d0 = idx_dma(0, 0)
    d0.start()
    d0.wait()

    def super_loop(sp, carry):
        q = lax.rem(sp, 2)
        qn = lax.rem(sp + 1, 2)

        @pl.when(sp > 0)
        def _():
            idx_dma(sp, q).wait()
            for p in range(HGROUP):
                sc_dma(q, p).wait()

        @pl.when(sp < nsuper - 1)
        def _():
            idx_dma(sp + 1, qn).start()
        for p in range(HGROUP):
            pltpu.async_copy(ones, hist.at[ibuf.at[q, p]], ssem, add=True)
        return carry

    lax.fori_loop(0, nsuper, super_loop, 0)
    for p in range(HGROUP):
        sc_dma(lax.rem(nsuper - 1, 2), p).wait()
    plsc.subcore_barrier()
    pltpu.sync_copy(hist.at[pl.ds(z0, HB // NSUB)], bounce)
    pltpu.sync_copy(bounce, out.at[pl.ds(c * HB + z0, HB // NSUB)])


def _hist_call(zh, ones_h, hidx):
    return pl.kernel(
        _hist_body,
        out_type=jax.ShapeDtypeStruct((2 * HB,), jnp.float32),
        mesh=_mesh(),
        scratch_types=[
            pltpu.VMEM_SHARED((HB,), jnp.float32),
            pltpu.VMEM((2, HGROUP, HCHUNK), jnp.int32),
            pltpu.VMEM((HCHUNK,), jnp.float32),
            pltpu.VMEM((HB // NSUB,), jnp.float32),
            pltpu.SemaphoreType.DMA,
            pltpu.SemaphoreType.DMA,
        ],
    )(zh, ones_h, hidx)


def _seg_body(z, ya, yb, ga, sa, gb, sb, outa, outb,
              agga, aggb, iga, isa, igb, isb, rowsa, rowsb,
              isem, ga0, ga1, gb0, gb1, sa0, sa1, sb0, sb1):
    c = lax.axis_index("c")
    s = lax.axis_index("s")
    w = c * NSUB + s
    z0 = s * (NAGG // NSUB)
    wrow = w * W_CH

    gsem = ((ga0, ga1), (gb0, gb1))
    ssem = ((sa0, sa1), (sb0, sb1))
    rows = (rowsa, rowsb)
    gi = (iga, igb)
    si = (isa, isb)
    ghbm = (ga, gb)
    shbm = (sa, sb)
    yy = (ya, yb)
    agg = (agga, aggb)

    def idx_copies(sp, q):
        rb = wrow + sp * SUPER
        return [pltpu.make_async_copy(ghbm[d].at[pl.ds(rb, SUPER)], gi[d].at[q], isem)
                for d in range(2)] + \
               [pltpu.make_async_copy(shbm[d].at[pl.ds(rb, SUPER)], si[d].at[q], isem)
                for d in range(2)]

    def fire_gather(d, q, k, p):
        return pltpu.async_copy(yy[d].at[gi[d].at[q, k]], rows[d].at[p],
                                gsem[d][p])

    def wait_gather(d, q, k, p):
        pltpu.make_async_copy(yy[d].at[gi[d].at[q, k]], rows[d].at[p],
                              gsem[d][p]).wait()

    def fire_scatter(d, q, k, p):
        return pltpu.async_copy(rows[d].at[p], agg[d].at[si[d].at[q, k]],
                                ssem[d][p], add=True)

    def wait_scatter(d, q, p):
        pltpu.make_async_copy(rows[d].at[p], agg[d].at[si[d].at[q, 0]],
                              ssem[d][p]).wait()

    for dsc in idx_copies(0, 0):
        dsc.start()
    pltpu.sync_copy(z.at[pl.ds(z0, NAGG // NSUB)], agga.at[pl.ds(z0, NAGG // NSUB)])
    pltpu.sync_copy(z.at[pl.ds(z0, NAGG // NSUB)], aggb.at[pl.ds(z0, NAGG // NSUB)])
    plsc.subcore_barrier()
    for dsc in idx_copies(0, 0):
        dsc.wait()

    def outer(sp, carry):
        q = lax.rem(sp, 2)
        qn = lax.rem(sp + 1, 2)

        @pl.when(sp > 0)
        def _():
            for dsc in idx_copies(sp, q):
                dsc.wait()

        @pl.when(sp < NSUPER - 1)
        def _():
            for dsc in idx_copies(sp + 1, qn):
                dsc.start()

        def inner(kk, icarry):
            first = jnp.logical_and(sp == 0, kk == 0)
            for p in range(2):
                @pl.when(jnp.logical_not(first))
                def _():
                    for d in range(2):
                        wait_scatter(d, q, p)
                k = 2 * kk + p
                for d in range(2):
                    fire_gather(d, q, k, p)
            for p in range(2):
                k = 2 * kk + p
                for d in range(2):
                    wait_gather(d, q, k, p)
                    fire_scatter(d, q, k, p)
            return icarry

        lax.fori_loop(0, SUPER // 2, inner, 0)
        return carry

    lax.fori_loop(0, NSUPER, outer, 0)
    for d in range(2):
        wait_scatter(d, (NSUPER - 1) % 2, 0)
        wait_scatter(d, (NSUPER - 1) % 2, 1)
    plsc.subcore_barrier()
    b0 = s * 200
    pltpu.sync_copy(agga.at[pl.ds(b0, 200)], outa.at[pl.ds(c * NU + b0, 200)])
    pltpu.sync_copy(aggb.at[pl.ds(b0, 200)], outb.at[pl.ds(c * NU + b0, 200)])

    @pl.when(s < 9)
    def _():
        b1 = (s + NSUB) * 200
        pltpu.sync_copy(agga.at[pl.ds(b1, 200)], outa.at[pl.ds(c * NU + b1, 200)])
        pltpu.sync_copy(aggb.at[pl.ds(b1, 200)], outb.at[pl.ds(c * NU + b1, 200)])


def _seg_call(z, ya, yb, ga, sa, gb, sb):
    return pl.kernel(
        _seg_body,
        out_type=(jax.ShapeDtypeStruct((2 * NU, H), jnp.float32),
                  jax.ShapeDtypeStruct((2 * NU, H), jnp.float32)),
        mesh=_mesh(),
        scratch_types=[
            pltpu.VMEM_SHARED((NAGG, H), jnp.float32),
            pltpu.VMEM_SHARED((NAGG, H), jnp.float32),
            pltpu.VMEM((2, SUPER, CHUNK), jnp.int32),
            pltpu.VMEM((2, SUPER, CHUNK), jnp.int32),
            pltpu.VMEM((2, SUPER, CHUNK), jnp.int32),
            pltpu.VMEM((2, SUPER, CHUNK), jnp.int32),
            pltpu.VMEM((2, CHUNK, H), jnp.float32),
            pltpu.VMEM((2, CHUNK, H), jnp.float32),
        ] + [pltpu.SemaphoreType.DMA] * 9,
    )(z, ya, yb, ga, sa, gb, sb)


def _scales_body(h_ref, o_ref):
    h = h_ref[...]
    one = jnp.float32(1.0)
    du = h[0:40] + h[120:160]
    ds = h[40:80] + h[160:200]
    cu = h[80:120] + h[200:240]
    o_ref[0:40] = lax.rsqrt(jnp.maximum(du, one))
    o_ref[40:80] = lax.rsqrt(jnp.maximum(ds, one))
    o_ref[80:120] = one / jnp.maximum(cu, one)
    o_ref[120:160] = one / jnp.maximum(ds, one)


def _scales_call(hist):
    return pl.pallas_call(
        _scales_body,
        out_shape=jax.ShapeDtypeStruct((160, 128), jnp.float32),
    )(hist)


_BLK = 1000
_GRID = NU // _BLK


def _rowspec():
    return pl.BlockSpec((_BLK, H), lambda i: (i, 0))


def _colspec():
    return pl.BlockSpec((_BLK, 1), lambda i: (i, 0))


def _pspec():
    return pl.BlockSpec((2, _BLK, H), lambda i: (0, i, 0))


def _wspec():
    return pl.BlockSpec((H, H), lambda i: (0, 0))


def _lgcn_pre_body(u_ref, s_ref, ru_ref, rs_ref, ut_ref, st_ref):
    ut_ref[...] = u_ref[...] * ru_ref[...]
    st_ref[...] = s_ref[...] * rs_ref[...]


def _lgcn_pre_call(u, s, ru, rs):
    return pl.pallas_call(
        _lgcn_pre_body,
        grid=(_GRID,),
        in_specs=[_rowspec(), _rowspec(), _colspec(), _colspec()],
        out_specs=(_rowspec(), _rowspec()),
        out_shape=(jax.ShapeDtypeStruct((NU, H), jnp.float32),
                   jax.ShapeDtypeStruct((NS, H), jnp.float32)),
    )(u, s, ru, rs)


def _lgcn_post1_body(pu_ref, ps_ref, u0_ref, s0_ref, ru_ref, rs_ref,
                     au_ref, as_ref, ut_ref, st_ref):
    ru = ru_ref[...]
    rs = rs_ref[...]
    uo = (pu_ref[0] + pu_ref[1]) * ru
    so = (ps_ref[0] + ps_ref[1]) * rs
    au_ref[...] = u0_ref[...] + uo
    as_ref[...] = s0_ref[...] + so
    ut_ref[...] = uo * ru
    st_ref[...] = so * rs


def _lgcn_post1_call(pu, ps, u0, s0, ru, rs):
    return pl.pallas_call(
        _lgcn_post1_body,
        grid=(_GRID,),
        in_specs=[_pspec(), _pspec(), _rowspec(), _rowspec(),
                  _colspec(), _colspec()],
        out_specs=(_rowspec(), _rowspec(), _rowspec(), _rowspec()),
        out_shape=tuple(jax.ShapeDtypeStruct((NU, H), jnp.float32)
                        for _ in range(4)),
    )(pu, ps, u0, s0, ru, rs)


def _lgcn_post2_body(pu_ref, ps_ref, au_ref, as_ref, ru_ref, rs_ref,
                     cu_ref, cs_ref):
    third = jnp.float32(1.0 / 3.0)
    cu_ref[...] = (au_ref[...] + (pu_ref[0] + pu_ref[1]) * ru_ref[...]) * third
    cs_ref[...] = (as_ref[...] + (ps_ref[0] + ps_ref[1]) * rs_ref[...]) * third


def _lgcn_post2_call(pu, ps, au, as_, ru, rs):
    return pl.pallas_call(
        _lgcn_post2_body,
        grid=(_GRID,),
        in_specs=[_pspec(), _pspec(), _rowspec(), _rowspec(),
                  _colspec(), _colspec()],
        out_specs=(_rowspec(), _rowspec()),
        out_shape=(jax.ShapeDtypeStruct((NU, H), jnp.float32),
                   jax.ShapeDtypeStruct((NS, H), jnp.float32)),
    )(pu, ps, au, as_, ru, rs)


def _mm_body(xu_ref, xs_ref, wus_ref, wsu_ref, wfu_ref, wfs_ref,
             yu_ref, ys_ref, su_ref, ss_ref):
    xu = xu_ref[...]
    xs = xs_ref[...]
    dn = (((1,), (0,)), ((), ()))
    yu_ref[...] = lax.dot_general(xu, wus_ref[...], dn,
                                  preferred_element_type=jnp.float32)
    ys_ref[...] = lax.dot_general(xs, wsu_ref[...], dn,
                                  preferred_element_type=jnp.float32)
    su_ref[...] = lax.dot_general(xu, wfu_ref[...], dn,
                                  preferred_element_type=jnp.float32)
    ss_ref[...] = lax.dot_general(xs, wfs_ref[...], dn,
                                  preferred_element_type=jnp.float32)


def _mm_call(xu, xs, wus, wsu, wfu, wfs):
    return pl.pallas_call(
        _mm_body,
        grid=(_GRID,),
        in_specs=[_rowspec(), _rowspec(), _wspec(), _wspec(), _wspec(), _wspec()],
        out_specs=tuple(_rowspec() for _ in range(4)),
        out_shape=tuple(jax.ShapeDtypeStruct((NU, H), jnp.float32)
                        for _ in range(4)),
    )(xu, xs, wus, wsu, wfu, wfs)


def _postmm_body(su0_ref, ss0_ref, pu_ref, ps_ref, icu_ref, ics_ref,
                 wus_ref, wsu_ref, wfu_ref, wfs_ref,
                 yu_ref, ys_ref, su_ref, ss_ref):
    zero = jnp.float32(0.0)
    xu = jnp.maximum(su0_ref[...] + (pu_ref[0] + pu_ref[1]) * icu_ref[...], zero)
    xs = jnp.maximum(ss0_ref[...] + (ps_ref[0] + ps_ref[1]) * ics_ref[...], zero)
    dn = (((1,), (0,)), ((), ()))
    yu_ref[...] = lax.dot_general(xu, wus_ref[...], dn,
                                  preferred_element_type=jnp.float32)
    ys_ref[...] = lax.dot_general(xs, wsu_ref[...], dn,
                                  preferred_element_type=jnp.float32)
    su_ref[...] = lax.dot_general(xu, wfu_ref[...], dn,
                                  preferred_element_type=jnp.float32)
    ss_ref[...] = lax.dot_general(xs, wfs_ref[...], dn,
                                  preferred_element_type=jnp.float32)


def _postmm_call(su0, ss0, pu, ps, icu, ics, wus, wsu, wfu, wfs):
    return pl.pallas_call(
        _postmm_body,
        grid=(_GRID,),
        in_specs=[_rowspec(), _rowspec(), _pspec(), _pspec(),
                  _colspec(), _colspec(),
                  _wspec(), _wspec(), _wspec(), _wspec()],
        out_specs=tuple(_rowspec() for _ in range(4)),
        out_shape=tuple(jax.ShapeDtypeStruct((NU, H), jnp.float32)
                        for _ in range(4)),
    )(su0, ss0, pu, ps, icu, ics, wus, wsu, wfu, wfs)


def _final_body(su_ref, ss_ref, pu_ref, ps_ref, icu_ref, ics_ref,
                ucat_ref, scat_ref, uo_ref, so_ref):
    zero = jnp.float32(0.0)
    uo_ref[:, 0:H] = ucat_ref[...]
    so_ref[:, 0:H] = scat_ref[...]
    uo_ref[:, H:2 * H] = jnp.maximum(
        su_ref[...] + (pu_ref[0] + pu_ref[1]) * icu_ref[...], zero)
    so_ref[:, H:2 * H] = jnp.maximum(
        ss_ref[...] + (ps_ref[0] + ps_ref[1]) * ics_ref[...], zero)


def _final_call(su, ss, pu, ps, icu, ics, ucat, scat):
    wide = pl.BlockSpec((_BLK, 2 * H), lambda i: (i, 0))
    return pl.pallas_call(
        _final_body,
        grid=(_GRID,),
        in_specs=[_rowspec(), _rowspec(), _pspec(), _pspec(),
                  _colspec(), _colspec(), _rowspec(), _rowspec()],
        out_specs=(wide, wide),
        out_shape=(jax.ShapeDtypeStruct((NU, 2 * H), jnp.float32),
                   jax.ShapeDtypeStruct((NS, 2 * H), jnp.float32)),
    )(su, ss, pu, ps, icu, ics, ucat, scat)


def _pad_gather(idx):
    pad = (jnp.arange(PADN, dtype=jnp.int32) * 13) % NU
    return jnp.concatenate([idx.astype(jnp.int32), pad]).reshape(ROWS_CH, CHUNK)


def _pad_scatter(idx):
    pad = NU + (jnp.arange(PADN, dtype=jnp.int32) % (NAGG - NU))
    return jnp.concatenate([idx.astype(jnp.int32), pad]).reshape(ROWS_CH, CHUNK)


def _hist_idx(idx, h):
    pad = h * NAGG + NU + (jnp.arange(PADN, dtype=jnp.int32) % (NAGG - NU))
    return jnp.concatenate([idx.astype(jnp.int32) + h * NAGG, pad])


def kernel(user_x, spot_x, lgcn_user, lgcn_spot, edge_index_us, edge_index_su,
           W0_self_u, W0_self_s, W0_su, W0_us,
           W1_self_u, W1_self_s, W1_su, W1_us,
           W2_self_u, W2_self_s, W2_su, W2_us):
    z = jnp.zeros((NAGG, H), jnp.float32)
    zh = jnp.zeros((HB,), jnp.float32)
    ones_h = jnp.ones((HCHUNK,), jnp.float32)

    us0 = edge_index_us[0]
    us1 = edge_index_us[1]
    su0 = edge_index_su[0]
    su1 = edge_index_su[1]

    us_g0 = _pad_gather(us0)
    us_g1 = _pad_gather(us1)
    us_s0 = _pad_scatter(us0)
    us_s1 = _pad_scatter(us1)
    su_g0 = _pad_gather(su0)
    su_s1 = _pad_scatter(su1)

    hidx = jnp.concatenate([
        _hist_idx(us0, 0), _hist_idx(us1, 1), _hist_idx(su1, 2),
    ]).reshape(HEP // HCHUNK, HCHUNK)

    hist = _hist_call(zh, ones_h, hidx)
    scales = _scales_call(hist.reshape(240, 128))
    sc = scales.reshape(4, NAGG)[:, :NU]
    ru = sc[0][:, None]
    rs = sc[1][:, None]
    icu = sc[2][:, None]
    ics = sc[3][:, None]

    ut0, st0 = _lgcn_pre_call(lgcn_user, lgcn_spot, ru, rs)
    pa, pb = _seg_call(z, st0, ut0, us_g1, us_s0, us_g0, us_s1)
    acc_u, acc_s, ut1, st1 = _lgcn_post1_call(
        pa.reshape(2, NU, H), pb.reshape(2, NS, H), lgcn_user, lgcn_spot, ru, rs)
    pa, pb = _seg_call(z, st1, ut1, us_g1, us_s0, us_g0, us_s1)
    u_cat, s_cat = _lgcn_post2_call(
        pa.reshape(2, NU, H), pb.reshape(2, NS, H), acc_u, acc_s, ru, rs)

    yu, ys, su_t, ss_t = _mm_call(user_x, spot_x, W0_us, W0_su,
                                  W0_self_u, W0_self_s)
    ps_, pu_ = _seg_call(z, yu, ys, us_g0, us_s1, su_g0, su_s1)
    yu, ys, su_t, ss_t = _postmm_call(
        su_t, ss_t, pu_.reshape(2, NU, H), ps_.reshape(2, NS, H), icu, ics,
        W1_us, W1_su, W1_self_u, W1_self_s)
    ps_, pu_ = _seg_call(z, yu, ys, us_g0, us_s1, su_g0, su_s1)
    yu, ys, su_t, ss_t = _postmm_call(
        su_t, ss_t, pu_.reshape(2, NU, H), ps_.reshape(2, NS, H), icu, ics,
        W2_us, W2_su, W2_self_u, W2_self_s)
    ps_, pu_ = _seg_call(z, yu, ys, us_g0, us_s1, su_g0, su_s1)
    return _final_call(su_t, ss_t, pu_.reshape(2, NU, H),
                       ps_.reshape(2, NS, H), icu, ics, u_cat, s_cat)


# speedup vs baseline: 1.0128x; 1.0012x over previous
import jax
import jax.numpy as jnp
from jax import lax
from jax.experimental import pallas as pl
from jax.experimental.pallas import tpu as pltpu
from jax.experimental.pallas import tpu_sc as plsc

NU = 5000
NS = 5000
D = 128
H = 128
E = 320000

NCORE = 2
NSUB = 16
CHUNK = 64
SUPER = 16
EP = 327680
PADN = EP - E
ROWS_CH = EP // CHUNK
W_CH = ROWS_CH // (NCORE * NSUB)
NSUPER = W_CH // SUPER
NAGG = 5120
HB = 3 * NAGG
HCHUNK = 128
HGROUP = 4
HEP = 3 * EP


def _mesh():
    return plsc.VectorSubcoreMesh(
        core_axis_name="c", subcore_axis_name="s",
        num_cores=NCORE, num_subcores=NSUB)


def _hist_body(zh, ones_h, hidx, out, hist, ibuf, ones, bounce, ssem, isem):
    c = lax.axis_index("c")
    s = lax.axis_index("s")
    w = c * NSUB + s
    z0 = s * (HB // NSUB)
    w_rows = HEP // HCHUNK // (NCORE * NSUB)
    wrow = w * w_rows
    nsuper = w_rows // HGROUP

    def idx_dma(sp, q):
        return pltpu.make_async_copy(
            hidx.at[pl.ds(wrow + sp * HGROUP, HGROUP)], ibuf.at[q], isem)

    def sc_dma(q, p):
        return pltpu.make_async_copy(ones, hist.at[ibuf.at[q, p]], ssem)

    d0 = idx_dma(0, 0)
    d0.start()
    pltpu.sync_copy(zh.at[pl.ds(z0, HB // NSUB)], bounce)
    pltpu.sync_copy(bounce, hist.at[pl.ds(z0, HB // NSUB)])
    pltpu.sync_copy(ones_h, ones)
    plsc.subcore_barrier()
    idx_dma(0, 0).wait()

    def super_loop(sp, carry):
        q = lax.rem(sp, 2)
        qn = lax.rem(sp + 1, 2)

        @pl.when(sp > 0)
        def _():
            idx_dma(sp, q).wait()
            for p in range(HGROUP):
                sc_dma(q, p).wait()

        @pl.when(sp < nsuper - 1)
        def _():
            idx_dma(sp + 1, qn).start()
        for p in range(HGROUP):
            pltpu.async_copy(ones, hist.at[ibuf.at[q, p]], ssem, add=True)
        return carry

    lax.fori_loop(0, nsuper, super_loop, 0)
    for p in range(HGROUP):
        sc_dma(lax.rem(nsuper - 1, 2), p).wait()
    plsc.subcore_barrier()
    pltpu.sync_copy(hist.at[pl.ds(z0, HB // NSUB)], bounce)
    pltpu.sync_copy(bounce, out.at[pl.ds(c * HB + z0, HB // NSUB)])


def _hist_call(zh, ones_h, hidx):
    return pl.kernel(
        _hist_body,
        out_type=jax.ShapeDtypeStruct((2 * HB,), jnp.float32),
        mesh=_mesh(),
        scratch_types=[
            pltpu.VMEM_SHARED((HB,), jnp.float32),
            pltpu.VMEM((2, HGROUP, HCHUNK), jnp.int32),
            pltpu.VMEM((HCHUNK,), jnp.float32),
            pltpu.VMEM((HB // NSUB,), jnp.float32),
            pltpu.SemaphoreType.DMA,
            pltpu.SemaphoreType.DMA,
        ],
    )(zh, ones_h, hidx)


def _seg_body(z, ya, yb, ga, sa, gb, sb, outa, outb,
              agga, aggb, iga, isa, igb, isb, rowsa, rowsb,
              isem, ga0, ga1, gb0, gb1, sa0, sa1, sb0, sb1):
    c = lax.axis_index("c")
    s = lax.axis_index("s")
    w = c * NSUB + s
    z0 = s * (NAGG // NSUB)
    wrow = w * W_CH

    gsem = ((ga0, ga1), (gb0, gb1))
    ssem = ((sa0, sa1), (sb0, sb1))
    rows = (rowsa, rowsb)
    gi = (iga, igb)
    si = (isa, isb)
    ghbm = (ga, gb)
    shbm = (sa, sb)
    yy = (ya, yb)
    agg = (agga, aggb)

    def idx_copies(sp, q):
        rb = wrow + sp * SUPER
        return [pltpu.make_async_copy(ghbm[d].at[pl.ds(rb, SUPER)], gi[d].at[q], isem)
                for d in range(2)] + \
               [pltpu.make_async_copy(shbm[d].at[pl.ds(rb, SUPER)], si[d].at[q], isem)
                for d in range(2)]

    def fire_gather(d, q, k, p):
        return pltpu.async_copy(yy[d].at[gi[d].at[q, k]], rows[d].at[p],
                                gsem[d][p])

    def wait_gather(d, q, k, p):
        pltpu.make_async_copy(yy[d].at[gi[d].at[q, k]], rows[d].at[p],
                              gsem[d][p]).wait()

    def fire_scatter(d, q, k, p):
        return pltpu.async_copy(rows[d].at[p], agg[d].at[si[d].at[q, k]],
                                ssem[d][p], add=True)

    def wait_scatter(d, q, p):
        pltpu.make_async_copy(rows[d].at[p], agg[d].at[si[d].at[q, 0]],
                              ssem[d][p]).wait()

    for dsc in idx_copies(0, 0):
        dsc.start()
    pltpu.sync_copy(z.at[pl.ds(z0, NAGG // NSUB)], agga.at[pl.ds(z0, NAGG // NSUB)])
    pltpu.sync_copy(z.at[pl.ds(z0, NAGG // NSUB)], aggb.at[pl.ds(z0, NAGG // NSUB)])
    plsc.subcore_barrier()
    for dsc in idx_copies(0, 0):
        dsc.wait()

    def outer(sp, carry):
        q = lax.rem(sp, 2)
        qn = lax.rem(sp + 1, 2)

        @pl.when(sp > 0)
        def _():
            for dsc in idx_copies(sp, q):
                dsc.wait()

        @pl.when(sp < NSUPER - 1)
        def _():
            for dsc in idx_copies(sp + 1, qn):
                dsc.start()

        def inner(kk, icarry):
            first = jnp.logical_and(sp == 0, kk == 0)
            for p in range(2):
                @pl.when(jnp.logical_not(first))
                def _():
                    for d in range(2):
                        wait_scatter(d, q, p)
                k = 2 * kk + p
                for d in range(2):
                    fire_gather(d, q, k, p)
            for p in range(2):
                k = 2 * kk + p
                for d in range(2):
                    wait_gather(d, q, k, p)
                    fire_scatter(d, q, k, p)
            return icarry

        lax.fori_loop(0, SUPER // 2, inner, 0)
        return carry

    lax.fori_loop(0, NSUPER, outer, 0)
    for d in range(2):
        wait_scatter(d, (NSUPER - 1) % 2, 0)
        wait_scatter(d, (NSUPER - 1) % 2, 1)
    plsc.subcore_barrier()
    b0 = s * 200
    pltpu.sync_copy(agga.at[pl.ds(b0, 200)], outa.at[pl.ds(c * NU + b0, 200)])
    pltpu.sync_copy(aggb.at[pl.ds(b0, 200)], outb.at[pl.ds(c * NU + b0, 200)])

    @pl.when(s < 9)
    def _():
        b1 = (s + NSUB) * 200
        pltpu.sync_copy(agga.at[pl.ds(b1, 200)], outa.at[pl.ds(c * NU + b1, 200)])
        pltpu.sync_copy(aggb.at[pl.ds(b1, 200)], outb.at[pl.ds(c * NU + b1, 200)])


def _seg_call(z, ya, yb, ga, sa, gb, sb):
    return pl.kernel(
        _seg_body,
        out_type=(jax.ShapeDtypeStruct((2 * NU, H), jnp.float32),
                  jax.ShapeDtypeStruct((2 * NU, H), jnp.float32)),
        mesh=_mesh(),
        scratch_types=[
            pltpu.VMEM_SHARED((NAGG, H), jnp.float32),
            pltpu.VMEM_SHARED((NAGG, H), jnp.float32),
            pltpu.VMEM((2, SUPER, CHUNK), jnp.int32),
            pltpu.VMEM((2, SUPER, CHUNK), jnp.int32),
            pltpu.VMEM((2, SUPER, CHUNK), jnp.int32),
            pltpu.VMEM((2, SUPER, CHUNK), jnp.int32),
            pltpu.VMEM((2, CHUNK, H), jnp.float32),
            pltpu.VMEM((2, CHUNK, H), jnp.float32),
        ] + [pltpu.SemaphoreType.DMA] * 9,
    )(z, ya, yb, ga, sa, gb, sb)


def _scales_body(h_ref, o_ref):
    h = h_ref[...]
    one = jnp.float32(1.0)
    du = h[0:40] + h[120:160]
    ds = h[40:80] + h[160:200]
    cu = h[80:120] + h[200:240]
    o_ref[0:40] = lax.rsqrt(jnp.maximum(du, one))
    o_ref[40:80] = lax.rsqrt(jnp.maximum(ds, one))
    o_ref[80:120] = one / jnp.maximum(cu, one)
    o_ref[120:160] = one / jnp.maximum(ds, one)


def _scales_call(hist):
    return pl.pallas_call(
        _scales_body,
        out_shape=jax.ShapeDtypeStruct((160, 128), jnp.float32),
    )(hist)


_BLK = 1000
_GRID = NU // _BLK


def _rowspec():
    return pl.BlockSpec((_BLK, H), lambda i: (i, 0))


def _colspec():
    return pl.BlockSpec((_BLK, 1), lambda i: (i, 0))


def _pspec():
    return pl.BlockSpec((2, _BLK, H), lambda i: (0, i, 0))


def _wspec():
    return pl.BlockSpec((H, H), lambda i: (0, 0))


def _lgcn_pre_body(u_ref, s_ref, ru_ref, rs_ref, ut_ref, st_ref):
    ut_ref[...] = u_ref[...] * ru_ref[...]
    st_ref[...] = s_ref[...] * rs_ref[...]


def _lgcn_pre_call(u, s, ru, rs):
    return pl.pallas_call(
        _lgcn_pre_body,
        grid=(_GRID,),
        in_specs=[_rowspec(), _rowspec(), _colspec(), _colspec()],
        out_specs=(_rowspec(), _rowspec()),
        out_shape=(jax.ShapeDtypeStruct((NU, H), jnp.float32),
                   jax.ShapeDtypeStruct((NS, H), jnp.float32)),
    )(u, s, ru, rs)


def _lgcn_post1_body(pu_ref, ps_ref, u0_ref, s0_ref, ru_ref, rs_ref,
                     au_ref, as_ref, ut_ref, st_ref):
    ru = ru_ref[...]
    rs = rs_ref[...]
    uo = (pu_ref[0] + pu_ref[1]) * ru
    so = (ps_ref[0] + ps_ref[1]) * rs
    au_ref[...] = u0_ref[...] + uo
    as_ref[...] = s0_ref[...] + so
    ut_ref[...] = uo * ru
    st_ref[...] = so * rs


def _lgcn_post1_call(pu, ps, u0, s0, ru, rs):
    return pl.pallas_call(
        _lgcn_post1_body,
        grid=(_GRID,),
        in_specs=[_pspec(), _pspec(), _rowspec(), _rowspec(),
                  _colspec(), _colspec()],
        out_specs=(_rowspec(), _rowspec(), _rowspec(), _rowspec()),
        out_shape=tuple(jax.ShapeDtypeStruct((NU, H), jnp.float32)
                        for _ in range(4)),
    )(pu, ps, u0, s0, ru, rs)


def _lgcn_post2_body(pu_ref, ps_ref, au_ref, as_ref, ru_ref, rs_ref,
                     cu_ref, cs_ref):
    third = jnp.float32(1.0 / 3.0)
    cu_ref[...] = (au_ref[...] + (pu_ref[0] + pu_ref[1]) * ru_ref[...]) * third
    cs_ref[...] = (as_ref[...] + (ps_ref[0] + ps_ref[1]) * rs_ref[...]) * third


def _lgcn_post2_call(pu, ps, au, as_, ru, rs):
    return pl.pallas_call(
        _lgcn_post2_body,
        grid=(_GRID,),
        in_specs=[_pspec(), _pspec(), _rowspec(), _rowspec(),
                  _colspec(), _colspec()],
        out_specs=(_rowspec(), _rowspec()),
        out_shape=(jax.ShapeDtypeStruct((NU, H), jnp.float32),
                   jax.ShapeDtypeStruct((NS, H), jnp.float32)),
    )(pu, ps, au, as_, ru, rs)


def _mm_body(xu_ref, xs_ref, wus_ref, wsu_ref, wfu_ref, wfs_ref,
             yu_ref, ys_ref, su_ref, ss_ref):
    xu = xu_ref[...]
    xs = xs_ref[...]
    dn = (((1,), (0,)), ((), ()))
    yu_ref[...] = lax.dot_general(xu, wus_ref[...], dn,
                                  preferred_element_type=jnp.float32)
    ys_ref[...] = lax.dot_general(xs, wsu_ref[...], dn,
                                  preferred_element_type=jnp.float32)
    su_ref[...] = lax.dot_general(xu, wfu_ref[...], dn,
                                  preferred_element_type=jnp.float32)
    ss_ref[...] = lax.dot_general(xs, wfs_ref[...], dn,
                                  preferred_element_type=jnp.float32)


def _mm_call(xu, xs, wus, wsu, wfu, wfs):
    return pl.pallas_call(
        _mm_body,
        grid=(_GRID,),
        in_specs=[_rowspec(), _rowspec(), _wspec(), _wspec(), _wspec(), _wspec()],
        out_specs=tuple(_rowspec() for _ in range(4)),
        out_shape=tuple(jax.ShapeDtypeStruct((NU, H), jnp.float32)
                        for _ in range(4)),
    )(xu, xs, wus, wsu, wfu, wfs)


def _postmm_body(su0_ref, ss0_ref, pu_ref, ps_ref, icu_ref, ics_ref,
                 wus_ref, wsu_ref, wfu_ref, wfs_ref,
                 yu_ref, ys_ref, su_ref, ss_ref):
    zero = jnp.float32(0.0)
    xu = jnp.maximum(su0_ref[...] + (pu_ref[0] + pu_ref[1]) * icu_ref[...], zero)
    xs = jnp.maximum(ss0_ref[...] + (ps_ref[0] + ps_ref[1]) * ics_ref[...], zero)
    dn = (((1,), (0,)), ((), ()))
    yu_ref[...] = lax.dot_general(xu, wus_ref[...], dn,
                                  preferred_element_type=jnp.float32)
    ys_ref[...] = lax.dot_general(xs, wsu_ref[...], dn,
                                  preferred_element_type=jnp.float32)
    su_ref[...] = lax.dot_general(xu, wfu_ref[...], dn,
                                  preferred_element_type=jnp.float32)
    ss_ref[...] = lax.dot_general(xs, wfs_ref[...], dn,
                                  preferred_element_type=jnp.float32)


def _postmm_call(su0, ss0, pu, ps, icu, ics, wus, wsu, wfu, wfs):
    return pl.pallas_call(
        _postmm_body,
        grid=(_GRID,),
        in_specs=[_rowspec(), _rowspec(), _pspec(), _pspec(),
                  _colspec(), _colspec(),
                  _wspec(), _wspec(), _wspec(), _wspec()],
        out_specs=tuple(_rowspec() for _ in range(4)),
        out_shape=tuple(jax.ShapeDtypeStruct((NU, H), jnp.float32)
                        for _ in range(4)),
    )(su0, ss0, pu, ps, icu, ics, wus, wsu, wfu, wfs)


def _final_body(su_ref, ss_ref, pu_ref, ps_ref, icu_ref, ics_ref,
                ucat_ref, scat_ref, uo_ref, so_ref):
    zero = jnp.float32(0.0)
    uo_ref[:, 0:H] = ucat_ref[...]
    so_ref[:, 0:H] = scat_ref[...]
    uo_ref[:, H:2 * H] = jnp.maximum(
        su_ref[...] + (pu_ref[0] + pu_ref[1]) * icu_ref[...], zero)
    so_ref[:, H:2 * H] = jnp.maximum(
        ss_ref[...] + (ps_ref[0] + ps_ref[1]) * ics_ref[...], zero)


def _final_call(su, ss, pu, ps, icu, ics, ucat, scat):
    wide = pl.BlockSpec((_BLK, 2 * H), lambda i: (i, 0))
    return pl.pallas_call(
        _final_body,
        grid=(_GRID,),
        in_specs=[_rowspec(), _rowspec(), _pspec(), _pspec(),
                  _colspec(), _colspec(), _rowspec(), _rowspec()],
        out_specs=(wide, wide),
        out_shape=(jax.ShapeDtypeStruct((NU, 2 * H), jnp.float32),
                   jax.ShapeDtypeStruct((NS, 2 * H), jnp.float32)),
    )(su, ss, pu, ps, icu, ics, ucat, scat)


def _pad_gather(idx):
    pad = (jnp.arange(PADN, dtype=jnp.int32) * 13) % NU
    return jnp.concatenate([idx.astype(jnp.int32), pad]).reshape(ROWS_CH, CHUNK)


def _pad_scatter(idx):
    pad = NU + (jnp.arange(PADN, dtype=jnp.int32) % (NAGG - NU))
    return jnp.concatenate([idx.astype(jnp.int32), pad]).reshape(ROWS_CH, CHUNK)


def _hist_idx(idx, h):
    pad = h * NAGG + NU + (jnp.arange(PADN, dtype=jnp.int32) % (NAGG - NU))
    return jnp.concatenate([idx.astype(jnp.int32) + h * NAGG, pad])


def kernel(user_x, spot_x, lgcn_user, lgcn_spot, edge_index_us, edge_index_su,
           W0_self_u, W0_self_s, W0_su, W0_us,
           W1_self_u, W1_self_s, W1_su, W1_us,
           W2_self_u, W2_self_s, W2_su, W2_us):
    z = jnp.zeros((NAGG, H), jnp.float32)
    zh = jnp.zeros((HB,), jnp.float32)
    ones_h = jnp.ones((HCHUNK,), jnp.float32)

    us0 = edge_index_us[0]
    us1 = edge_index_us[1]
    su0 = edge_index_su[0]
    su1 = edge_index_su[1]

    us_g0 = _pad_gather(us0)
    us_g1 = _pad_gather(us1)
    us_s0 = _pad_scatter(us0)
    us_s1 = _pad_scatter(us1)
    su_g0 = _pad_gather(su0)
    su_s1 = _pad_scatter(su1)

    hidx = jnp.concatenate([
        _hist_idx(us0, 0), _hist_idx(us1, 1), _hist_idx(su1, 2),
    ]).reshape(HEP // HCHUNK, HCHUNK)

    hist = _hist_call(zh, ones_h, hidx)
    scales = _scales_call(hist.reshape(240, 128))
    sc = scales.reshape(4, NAGG)[:, :NU]
    ru = sc[0][:, None]
    rs = sc[1][:, None]
    icu = sc[2][:, None]
    ics = sc[3][:, None]

    ut0, st0 = _lgcn_pre_call(lgcn_user, lgcn_spot, ru, rs)
    pa, pb = _seg_call(z, st0, ut0, us_g1, us_s0, us_g0, us_s1)
    acc_u, acc_s, ut1, st1 = _lgcn_post1_call(
        pa.reshape(2, NU, H), pb.reshape(2, NS, H), lgcn_user, lgcn_spot, ru, rs)
    pa, pb = _seg_call(z, st1, ut1, us_g1, us_s0, us_g0, us_s1)
    u_cat, s_cat = _lgcn_post2_call(
        pa.reshape(2, NU, H), pb.reshape(2, NS, H), acc_u, acc_s, ru, rs)

    yu, ys, su_t, ss_t = _mm_call(user_x, spot_x, W0_us, W0_su,
                                  W0_self_u, W0_self_s)
    ps_, pu_ = _seg_call(z, yu, ys, us_g0, us_s1, su_g0, su_s1)
    yu, ys, su_t, ss_t = _postmm_call(
        su_t, ss_t, pu_.reshape(2, NU, H), ps_.reshape(2, NS, H), icu, ics,
        W1_us, W1_su, W1_self_u, W1_self_s)
    ps_, pu_ = _seg_call(z, yu, ys, us_g0, us_s1, su_g0, su_s1)
    yu, ys, su_t, ss_t = _postmm_call(
        su_t, ss_t, pu_.reshape(2, NU, H), ps_.reshape(2, NS, H), icu, ics,
        W2_us, W2_su, W2_self_u, W2_self_s)
    ps_, pu_ = _seg_call(z, yu, ys, us_g0, us_s1, su_g0, su_s1)
    return _final_call(su_t, ss_t, pu_.reshape(2, NU, H),
                       ps_.reshape(2, NS, H), icu, ics, u_cat, s_cat)
